# baseline (speedup 1.0000x reference)
import jax
import jax.numpy as jnp
from jax.experimental import pallas as pl
from jax.experimental.pallas import tpu as pltpu

_E = 8


def _coeff(m_ref, rw_ref):
    m = m_ref[0]
    return (m[:, 0:1].astype(jnp.float32) * rw_ref[:, 0:1]
            + m[:, 1:2].astype(jnp.float32) * rw_ref[:, 1:2])


def _mm(a, w_ref):
    return jax.lax.dot_general(
        a, w_ref[0], (((1,), (1,)), ((), ())),
        preferred_element_type=jnp.float32,
    )


def _body(m1_ref, rw1_ref, x_ref, w1_ref, b1_ref,
          m2_ref, rw2_ref, w2_ref, b2_ref,
          m3_ref, rw3_ref, w3_ref, b3_ref,
          out_ref, cs1_ref, cs2_ref):
    g = pl.program_id(0)

    @pl.when(g < 2 * _E)
    def _stage1():
        kc = g // _E
        y = _mm(x_ref[...], w1_ref)
        y = y + b1_ref[0] * (kc == 0).astype(jnp.float32)
        contrib = y * _coeff(m1_ref, rw1_ref)

        @pl.when(g == 0)
        def _():
            cs1_ref[...] = contrib

        @pl.when(g > 0)
        def _():
            cs1_ref[...] = cs1_ref[...] + contrib

    @pl.when((g >= 2 * _E) & (g < 3 * _E))
    def _stage2():
        y = _mm(cs1_ref[...], w2_ref) + b2_ref[0]
        contrib = y * _coeff(m2_ref, rw2_ref)

        @pl.when(g == 2 * _E)
        def _():
            cs2_ref[...] = contrib

        @pl.when(g > 2 * _E)
        def _():
            cs2_ref[...] = cs2_ref[...] + contrib

    @pl.when(g >= 3 * _E)
    def _stage3():
        gg = g - 3 * _E
        e = gg % _E
        y = _mm(cs2_ref[...], w3_ref) + b3_ref[0]
        y = jnp.maximum(y, 0.0)
        contrib = y * _coeff(m3_ref, rw3_ref)

        @pl.when(e == 0)
        def _():
            out_ref[...] = contrib

        @pl.when(e > 0)
        def _():
            out_ref[...] = out_ref[...] + contrib

        @pl.when(e == _E - 1)
        def _():
            out_ref[...] = jnp.maximum(out_ref[...], 0.0)


def kernel(x, expert_mask1, expert_mask2, expert_mask3,
           routing_weights1, routing_weights2, routing_weights3,
           W1, b1, W2, b2, W3, b3):
    bsz, seq_len, hidden = x.shape
    T = bsz * seq_len
    xf = x.reshape(T, hidden)
    E, R0, H = W1.shape
    R1 = W2.shape[1]
    OUT = W3.shape[1]
    HC = H // 2
    NC = OUT // 4

    def e1(g):
        return jnp.where(g < 2 * _E, g % _E, _E - 1)

    def kc1(g):
        return jnp.clip(g // _E, 0, 1)

    def e2(g):
        return jnp.clip(g - 2 * _E, 0, _E - 1)

    def e3(g):
        return jnp.where(g >= 3 * _E, (g - 3 * _E) % _E, 0)

    def nc3(g):
        return jnp.clip((g - 3 * _E) // _E, 0, 3)

    out = pl.pallas_call(
        _body,
        grid=(7 * _E,),
        in_specs=[
            pl.BlockSpec((1, T, 2), lambda g: (e1(g), 0, 0)),
            pl.BlockSpec((T, 2), lambda g: (0, 0)),
            pl.BlockSpec((T, HC), lambda g: (0, kc1(g))),
            pl.BlockSpec((1, R0, HC), lambda g: (e1(g), 0, kc1(g))),
            pl.BlockSpec((1, 1, R0), lambda g: (e1(g), 0, 0)),
            pl.BlockSpec((1, T, 2), lambda g: (e2(g), 0, 0)),
            pl.BlockSpec((T, 2), lambda g: (0, 0)),
            pl.BlockSpec((1, R1, R0), lambda g: (e2(g), 0, 0)),
            pl.BlockSpec((1, 1, R1), lambda g: (e2(g), 0, 0)),
            pl.BlockSpec((1, T, 2), lambda g: (e3(g), 0, 0)),
            pl.BlockSpec((T, 2), lambda g: (0, 0)),
            pl.BlockSpec((1, NC, R1), lambda g: (e3(g), nc3(g), 0)),
            pl.BlockSpec((1, 1, NC), lambda g: (e3(g), 0, nc3(g))),
        ],
        out_specs=pl.BlockSpec((T, NC), lambda g: (0, nc3(g))),
        out_shape=jax.ShapeDtypeStruct((T, OUT), jnp.float32),
        scratch_shapes=[
            pltpu.VMEM((T, R0), jnp.float32),
            pltpu.VMEM((T, R1), jnp.float32),
        ],
        compiler_params=pltpu.CompilerParams(
            dimension_semantics=("arbitrary",),
        ),
    )(
        expert_mask1.transpose(0, 2, 1), routing_weights1, xf,
        W1, b1.reshape(E, 1, R0),
        expert_mask2.transpose(0, 2, 1), routing_weights2,
        W2, b2.reshape(E, 1, R1),
        expert_mask3.transpose(0, 2, 1), routing_weights3,
        W3, b3.reshape(E, 1, OUT),
    )
    return out.reshape(bsz, seq_len, OUT)


# speedup vs baseline: 1.0892x; 1.0892x over previous
import jax
import jax.numpy as jnp
from jax.experimental import pallas as pl
from jax.experimental.pallas import tpu as pltpu

_E = 8


def _coeff(m_ref, rw_ref):
    m = m_ref[0]
    rw = rw_ref[0]
    return (m[:, 0:1].astype(jnp.float32) * rw[:, 0:1]
            + m[:, 1:2].astype(jnp.float32) * rw[:, 1:2])


def _mm(a, w_ref):
    return jax.lax.dot_general(
        a, w_ref[0], (((1,), (1,)), ((), ())),
        preferred_element_type=jnp.float32,
    )


def _body(m_ref, rw_ref, x_ref, w1_ref, b1_ref, w2_ref, b2_ref,
          w3_ref, b3_ref, out_ref, cs1_ref, cs2_ref):
    g = pl.program_id(0)

    @pl.when(g < _E)
    def _stage1():
        y = _mm(x_ref[...], w1_ref) + b1_ref[0]
        contrib = y * _coeff(m_ref, rw_ref)

        @pl.when(g == 0)
        def _():
            cs1_ref[...] = contrib

        @pl.when(g > 0)
        def _():
            cs1_ref[...] = cs1_ref[...] + contrib

    @pl.when((g >= _E) & (g < 2 * _E))
    def _stage2():
        y = _mm(cs1_ref[...], w2_ref) + b2_ref[0]
        contrib = y * _coeff(m_ref, rw_ref)

        @pl.when(g == _E)
        def _():
            cs2_ref[...] = contrib

        @pl.when(g > _E)
        def _():
            cs2_ref[...] = cs2_ref[...] + contrib

    @pl.when(g >= 2 * _E)
    def _stage3():
        gg = g - 2 * _E
        e = gg % _E
        y = _mm(cs2_ref[...], w3_ref) + b3_ref[0]
        y = jnp.maximum(y, 0.0)
        contrib = y * _coeff(m_ref, rw_ref)

        @pl.when(e == 0)
        def _():
            out_ref[...] = contrib

        @pl.when(e > 0)
        def _():
            out_ref[...] = out_ref[...] + contrib

        @pl.when(e == _E - 1)
        def _():
            out_ref[...] = jnp.maximum(out_ref[...], 0.0)


def kernel(x, expert_mask1, expert_mask2, expert_mask3,
           routing_weights1, routing_weights2, routing_weights3,
           W1, b1, W2, b2, W3, b3):
    bsz, seq_len, hidden = x.shape
    T = bsz * seq_len
    xf = x.reshape(T, hidden)
    E, R0, H = W1.shape
    R1 = W2.shape[1]
    OUT = W3.shape[1]
    HC = H
    NC = OUT // 4

    def e1(g):
        return jnp.clip(g, 0, _E - 1)

    def e2(g):
        return jnp.clip(g - _E, 0, _E - 1)

    def e3(g):
        return jnp.where(g >= 2 * _E, (g - 2 * _E) % _E, 0)

    def nc3(g):
        return jnp.clip((g - 2 * _E) // _E, 0, 3)

    def midx(g):
        return jnp.where(g < 2 * _E, g, 2 * _E + (g - 2 * _E) % _E)

    def sidx(g):
        return jnp.where(g < _E, 0, jnp.where(g < 2 * _E, 1, 2))

    out = pl.pallas_call(
        _body,
        grid=(6 * _E,),
        in_specs=[
            pl.BlockSpec((1, T, 2), lambda g: (midx(g), 0, 0)),
            pl.BlockSpec((1, T, 2), lambda g: (sidx(g), 0, 0)),
            pl.BlockSpec((T, HC), lambda g: (0, 0)),
            pl.BlockSpec((1, R0, HC), lambda g: (e1(g), 0, 0)),
            pl.BlockSpec((1, 1, R0), lambda g: (e1(g), 0, 0)),
            pl.BlockSpec((1, R1, R0), lambda g: (e2(g), 0, 0)),
            pl.BlockSpec((1, 1, R1), lambda g: (e2(g), 0, 0)),
            pl.BlockSpec((1, NC, R1), lambda g: (e3(g), nc3(g), 0)),
            pl.BlockSpec((1, 1, NC), lambda g: (e3(g), 0, nc3(g))),
        ],
        out_specs=pl.BlockSpec((T, NC), lambda g: (0, nc3(g))),
        out_shape=jax.ShapeDtypeStruct((T, OUT), jnp.float32),
        scratch_shapes=[
            pltpu.VMEM((T, R0), jnp.float32),
            pltpu.VMEM((T, R1), jnp.float32),
        ],
        compiler_params=pltpu.CompilerParams(
            dimension_semantics=("arbitrary",),
        ),
    )(
        jnp.concatenate([expert_mask1.transpose(0, 2, 1),
                         expert_mask2.transpose(0, 2, 1),
                         expert_mask3.transpose(0, 2, 1)], axis=0),
        jnp.stack([routing_weights1, routing_weights2, routing_weights3]),
        xf, W1, b1.reshape(E, 1, R0), W2, b2.reshape(E, 1, R1),
        W3, b3.reshape(E, 1, OUT),
    )
    return out.reshape(bsz, seq_len, OUT)
